# baseline (speedup 1.0000x reference)
import jax
import jax.numpy as jnp
from jax import lax
from jax.experimental import pallas as pl
from jax.experimental.pallas import tpu as pltpu
from jax.experimental.pallas import tpu_sc as plsc

_NC, _NS = 2, 16
_NW = _NC * _NS
_LANE = 16


def _t3_body(t_ref, w_ref, b_ref, o_ref):
    e = jnp.maximum(t_ref[...].T, 0.0)
    h = lax.dot(e, w_ref[...], preferred_element_type=jnp.float32)
    o_ref[...] = jnp.maximum(h + b_ref[...], 0.0)


def _transform_table(table_t, phi_W, phi_b):
    d, n = table_t.shape
    h = phi_W.shape[1]
    rblk = 4096
    return pl.pallas_call(
        _t3_body,
        grid=(pl.cdiv(n, rblk),),
        in_specs=[pl.BlockSpec((d, rblk), lambda i: (0, i)),
                  pl.BlockSpec((d, h), lambda i: (0, 0)),
                  pl.BlockSpec((1, h), lambda i: (0, 0))],
        out_specs=pl.BlockSpec((rblk, h), lambda i: (i, 0)),
        out_shape=jax.ShapeDtypeStruct((n, h), jnp.float32),
    )(table_t, phi_W, phi_b.reshape(1, h))


_NBUF = 6


def _embedding_bag(t3, xcat):
    b, setl = xcat.shape
    n, h = t3.shape
    bpw = b // _NW
    pairs = bpw // 2
    idx2 = xcat.reshape(b // 2, 2 * setl)
    nvec = h // _LANE

    def body(t3_hbm, idx_hbm, out_hbm, idx_v, rows_v, acc_v, sem):
        wid = lax.axis_index("s") * _NC + lax.axis_index("c")
        pltpu.sync_copy(idx_hbm.at[pl.ds(wid * pairs, pairs)], idx_v)
        for k in range(_NBUF - 1):
            pltpu.async_copy(t3_hbm.at[idx_v.at[k]], rows_v.at[k], sem)

        def outer(i4, carry):
            for k in range(2):
                p = i4 * 2 + k
                kb = lax.rem(p, _NBUF)
                pltpu.make_async_copy(
                    t3_hbm.at[idx_v.at[p]], rows_v.at[kb], sem).wait()

                nxt = p + _NBUF - 1

                @pl.when(nxt < pairs)
                def _start_next():
                    pltpu.async_copy(
                        t3_hbm.at[idx_v.at[nxt]],
                        rows_v.at[lax.rem(nxt, _NBUF)], sem)

                for half in range(2):
                    acc = [rows_v[kb, half * setl, pl.ds(_LANE * j, _LANE)]
                           for j in range(nvec)]
                    for l in range(1, setl):
                        for j in range(nvec):
                            acc[j] = acc[j] + rows_v[
                                kb, half * setl + l, pl.ds(_LANE * j, _LANE)]
                    row = p * 2 + half
                    for j in range(nvec):
                        acc_v[row, pl.ds(_LANE * j, _LANE)] = acc[j]
            return carry

        lax.fori_loop(0, pairs // 2, outer, 0)
        pltpu.sync_copy(acc_v, out_hbm.at[pl.ds(wid * bpw, bpw)])

    kern = pl.kernel(
        body,
        out_type=jax.ShapeDtypeStruct((b, h), jnp.float32),
        mesh=plsc.VectorSubcoreMesh(core_axis_name="c", subcore_axis_name="s"),
        scratch_types=[
            pltpu.VMEM((pairs, 2 * setl), jnp.int32),
            pltpu.VMEM((_NBUF, 2 * setl, h), jnp.float32),
            pltpu.VMEM((bpw, h), jnp.float32),
            pltpu.SemaphoreType.DMA,
        ],
    )
    return kern(t3, idx2)


def _head_body(p_ref, rw_ref, rb_ref, fw_ref, fb_ref, o_ref):
    x = lax.dot(p_ref[...], rw_ref[...], preferred_element_type=jnp.float32)
    x = jnp.maximum(x + rb_ref[...], 0.0)
    o_ref[...] = jnp.sum(x * fw_ref[...], axis=1, keepdims=True) + fb_ref[...]


def _head(pooled, rho_W, rho_b, fc_W, fc_b):
    b, h = pooled.shape
    return pl.pallas_call(
        _head_body,
        out_shape=jax.ShapeDtypeStruct((b, 1), jnp.float32),
    )(pooled, rho_W, rho_b.reshape(1, h), fc_W.reshape(1, h),
      fc_b.reshape(1, 1))


def kernel(xcat, embed_table, phi_W, phi_b, rho_W, rho_b, fc_W, fc_b):
    t3 = _transform_table(embed_table.T, phi_W, phi_b)
    pooled = _embedding_bag(t3, xcat.astype(jnp.int32))
    return _head(pooled, rho_W, rho_b, fc_W, fc_b)


# speedup vs baseline: 1.9389x; 1.9389x over previous
import jax
import jax.numpy as jnp
from jax import lax
from jax.experimental import pallas as pl
from jax.experimental.pallas import tpu as pltpu
from jax.experimental.pallas import tpu_sc as plsc

_NC, _NS = 2, 16
_NW = _NC * _NS
_LANE = 16


def _t3_body(t_ref, w_ref, b_ref, o_ref):
    e = jnp.maximum(t_ref[...].T, 0.0)
    h = lax.dot(e, w_ref[...], preferred_element_type=jnp.float32)
    o_ref[...] = jnp.maximum(h + b_ref[...], 0.0)


def _transform_table(table_t, phi_W, phi_b):
    d, n = table_t.shape
    h = phi_W.shape[1]
    rblk = 4096
    return pl.pallas_call(
        _t3_body,
        grid=(pl.cdiv(n, rblk),),
        in_specs=[pl.BlockSpec((d, rblk), lambda i: (0, i)),
                  pl.BlockSpec((d, h), lambda i: (0, 0)),
                  pl.BlockSpec((1, h), lambda i: (0, 0))],
        out_specs=pl.BlockSpec((rblk, h), lambda i: (i, 0)),
        out_shape=jax.ShapeDtypeStruct((n, h), jnp.float32),
    )(table_t, phi_W, phi_b.reshape(1, h))


_NBUF = 6


def _embedding_bag(t3, xcat):
    b, setl = xcat.shape
    n, h = t3.shape
    bpw = b // _NW
    pairs = bpw // 2
    idx2 = xcat.reshape(b // 2, 2 * setl)
    nvec = h // _LANE

    def body(t3_hbm, idx_hbm, out_hbm, idx_v, rows_v, acc_v, sem):
        wid = lax.axis_index("s") * _NC + lax.axis_index("c")
        pltpu.sync_copy(idx_hbm.at[pl.ds(wid * pairs, pairs)], idx_v)
        for k in range(_NBUF - 1):
            pltpu.async_copy(t3_hbm.at[idx_v.at[k]], rows_v.at[k], sem)

        def outer(i4, carry):
            for k in range(2):
                p = i4 * 2 + k
                kb = lax.rem(p, _NBUF)
                pltpu.make_async_copy(
                    t3_hbm.at[idx_v.at[p]], rows_v.at[kb], sem).wait()

                nxt = p + _NBUF - 1

                @pl.when(nxt < pairs)
                def _start_next():
                    pltpu.async_copy(
                        t3_hbm.at[idx_v.at[nxt]],
                        rows_v.at[lax.rem(nxt, _NBUF)], sem)

                for half in range(2):
                    def inner(l2, acc, _kb=kb, _half=half):
                        for u in range(2):
                            acc = tuple(
                                acc[j] + rows_v[_kb,
                                                _half * setl + l2 * 2 + u,
                                                pl.ds(_LANE * j, _LANE)]
                                for j in range(nvec))
                        return acc
                    acc = lax.fori_loop(
                        0, setl // 2, inner,
                        tuple(jnp.zeros((_LANE,), jnp.float32)
                              for _ in range(nvec)))
                    row = p * 2 + half
                    for j in range(nvec):
                        acc_v[row, pl.ds(_LANE * j, _LANE)] = acc[j]
            return carry

        lax.fori_loop(0, pairs // 2, outer, 0)
        pltpu.sync_copy(acc_v, out_hbm.at[pl.ds(wid * bpw, bpw)])

    kern = pl.kernel(
        body,
        out_type=jax.ShapeDtypeStruct((b, h), jnp.float32),
        mesh=plsc.VectorSubcoreMesh(core_axis_name="c", subcore_axis_name="s"),
        scratch_types=[
            pltpu.VMEM((pairs, 2 * setl), jnp.int32),
            pltpu.VMEM((_NBUF, 2 * setl, h), jnp.float32),
            pltpu.VMEM((bpw, h), jnp.float32),
            pltpu.SemaphoreType.DMA,
        ],
    )
    return kern(t3, idx2)


def _head_body(p_ref, rw_ref, rb_ref, fw_ref, fb_ref, o_ref):
    x = lax.dot(p_ref[...], rw_ref[...], preferred_element_type=jnp.float32)
    x = jnp.maximum(x + rb_ref[...], 0.0)
    o_ref[...] = jnp.sum(x * fw_ref[...], axis=1, keepdims=True) + fb_ref[...]


def _head(pooled, rho_W, rho_b, fc_W, fc_b):
    b, h = pooled.shape
    return pl.pallas_call(
        _head_body,
        out_shape=jax.ShapeDtypeStruct((b, 1), jnp.float32),
    )(pooled, rho_W, rho_b.reshape(1, h), fc_W.reshape(1, h),
      fc_b.reshape(1, 1))


def kernel(xcat, embed_table, phi_W, phi_b, rho_W, rho_b, fc_W, fc_b):
    t3 = _transform_table(embed_table.T, phi_W, phi_b)
    pooled = _embedding_bag(t3, xcat.astype(jnp.int32))
    return _head(pooled, rho_W, rho_b, fc_W, fc_b)
